# baseline (speedup 1.0000x reference)
import functools

import jax
import jax.numpy as jnp
from jax import lax
from jax.experimental import pallas as pl
from jax.experimental.pallas import tpu as pltpu
from jax.experimental.pallas import tpu_sc as plsc

BATCH = 1024
SEQ = 200
D = 64
SCALE = 8.0

NC, NS, L = 2, 16, 16
NW = NC * NS
ROWS = BATCH * SEQ
RPW = ROWS // NW
CHUNK = 100
CHUNKS = RPW // CHUNK
NBUF = 4
GROUPS = CHUNKS // NBUF


def _body(tok_hbm, pe_hbm, table_hbm, out_hbm,
          idx_v, pe_v,
          in0, in1, in2, in3, ob0, ob1, ob2, ob3,
          si0, si1, si2, si3, so0, so1, so2, so3):
  ins = (in0, in1, in2, in3)
  obs = (ob0, ob1, ob2, ob3)
  sin = (si0, si1, si2, si3)
  sout = (so0, so1, so2, so3)

  wid = lax.axis_index("s") * NC + lax.axis_index("c")
  idx_base = wid * CHUNKS
  out_base = wid * RPW

  pltpu.sync_copy(tok_hbm.at[pl.ds(idx_base, CHUNKS)], idx_v)
  pltpu.sync_copy(pe_hbm, pe_v)

  def fire_gather(c, b):
    pltpu.async_copy(table_hbm.at[idx_v.at[c]], ins[b], sin[b])

  def wait_gather(c, b):
    pltpu.make_async_copy(table_hbm.at[idx_v.at[c]], ins[b], sin[b]).wait()

  def fire_out(c, b):
    pltpu.async_copy(obs[b],
                     out_hbm.at[pl.ds((out_base + c * CHUNK) * D, CHUNK * D)],
                     sout[b])

  def wait_out(c, b):
    pltpu.make_async_copy(
        obs[b], out_hbm.at[pl.ds((out_base + c * CHUNK) * D, CHUNK * D)],
        sout[b]).wait()

  def compute(b):
    peoff = (b % 2) * CHUNK
    src, dst = ins[b], obs[b]

    @plsc.parallel_loop(0, CHUNK, unroll=4)
    def _(r):
      for j in range(D // L):
        x = src[r, pl.ds(j * L, L)] * SCALE
        dst[pl.ds(r * D + j * L, L)] = x + pe_v[peoff + r, pl.ds(j * L, L)]

  for b in range(NBUF):
    fire_gather(b, b)

  for b in range(NBUF):
    wait_gather(b, b)
    compute(b)
    fire_out(b, b)
    fire_gather(NBUF + b, b)

  @pl.loop(1, GROUPS - 1)
  def _(g):
    for b in range(NBUF):
      c = g * NBUF + b
      wait_gather(c, b)
      wait_out(c - NBUF, b)
      compute(b)
      fire_out(c, b)
      fire_gather(c + NBUF, b)

  for b in range(NBUF):
    c = (GROUPS - 1) * NBUF + b
    wait_gather(c, b)
    wait_out(c - NBUF, b)
    compute(b)
    fire_out(c, b)

  for b in range(NBUF):
    wait_out((GROUPS - 1) * NBUF + b, b)


@jax.jit
def _embed(tokens2d, pe200, table):
  mesh = plsc.VectorSubcoreMesh(core_axis_name="c", subcore_axis_name="s")
  f = pl.kernel(
      _body,
      out_type=jax.ShapeDtypeStruct((ROWS * D,), jnp.float32),
      mesh=mesh,
      scratch_types=(
          [pltpu.VMEM((CHUNKS, CHUNK), jnp.int32),
           pltpu.VMEM((SEQ, D), jnp.float32)]
          + [pltpu.VMEM((CHUNK, D), jnp.float32) for _ in range(NBUF)]
          + [pltpu.VMEM((CHUNK * D,), jnp.float32) for _ in range(NBUF)]
          + [pltpu.SemaphoreType.DMA for _ in range(2 * NBUF)]
      ),
      compiler_params=pltpu.CompilerParams(use_tc_tiling_on_sc=False),
  )
  return f(tokens2d, pe200, table)


def kernel(tokens, table, pe):
  tokens2d = tokens.astype(jnp.int32).reshape(NW * CHUNKS, CHUNK)
  pe200 = pe[:SEQ]
  out = _embed(tokens2d, pe200, table)
  return out.reshape(BATCH, SEQ, D)
